# baseline (speedup 1.0000x reference)
import functools

import jax
import jax.numpy as jnp
from jax import lax
from jax.experimental import pallas as pl
from jax.experimental.pallas import tpu as pltpu
from jax.experimental.pallas import tpu_sc as plsc

_NUM_CORES = 2
_NUM_SUBCORES = 16
_NUM_WORKERS = _NUM_CORES * _NUM_SUBCORES


def _build_gather(B, D, dtype, b_per_w):
    mesh = plsc.VectorSubcoreMesh(core_axis_name="c", subcore_axis_name="s")

    @functools.partial(
        pl.kernel,
        out_type=jax.ShapeDtypeStruct((B, D), dtype),
        mesh=mesh,
        scratch_types=[
            pltpu.VMEM((b_per_w,), jnp.int32),
            pltpu.VMEM((b_per_w, D), dtype),
            pltpu.SemaphoreType.DMA,
        ],
    )
    def gather(table_hbm, idx_hbm, out_hbm, idx_v, rows_v, sem):
        wid = lax.axis_index("s") * _NUM_CORES + lax.axis_index("c")
        base = wid * b_per_w
        pltpu.sync_copy(idx_hbm.at[pl.ds(base, b_per_w)], idx_v)
        pltpu.async_copy(table_hbm.at[idx_v], rows_v, sem).wait()
        pltpu.sync_copy(rows_v, out_hbm.at[pl.ds(base, b_per_w)])

    return gather


def kernel(embeddings, indices):
    (B,) = indices.shape
    _, D = embeddings.shape
    Dp = 128
    b_per_w = B // _NUM_WORKERS
    table = jnp.pad(embeddings, ((0, 0), (0, Dp - D)))
    gather = _build_gather(B, Dp, embeddings.dtype, b_per_w)
    out = gather(table, indices.astype(jnp.int32))
    return out[:, :D]


# speedup vs baseline: 3.1356x; 3.1356x over previous
import functools

import jax
import jax.numpy as jnp
from jax import lax
from jax.experimental import pallas as pl
from jax.experimental.pallas import tpu as pltpu
from jax.experimental.pallas import tpu_sc as plsc

_NUM_CORES = 2
_NUM_SUBCORES = 16
_NUM_WORKERS = _NUM_CORES * _NUM_SUBCORES

_CHUNK = 16


def _build_gather(B, D, dtype, b_per_w):
    mesh = plsc.VectorSubcoreMesh(core_axis_name="c", subcore_axis_name="s")
    n_chunks = b_per_w // _CHUNK

    @functools.partial(
        pl.kernel,
        out_type=jax.ShapeDtypeStruct((B, D), dtype),
        mesh=mesh,
        scratch_types=[
            pltpu.VMEM((b_per_w,), jnp.int32),
            pltpu.VMEM((b_per_w, D), dtype),
            pltpu.SemaphoreType.DMA,
        ],
    )
    def gather(table_hbm, idx_hbm, out_hbm, idx_v, rows_v, sem_r):
        wid = lax.axis_index("s") * _NUM_CORES + lax.axis_index("c")
        base = wid * b_per_w
        pltpu.sync_copy(idx_hbm.at[pl.ds(base, b_per_w)], idx_v)

        def fire(c, carry):
            vec = idx_v[pl.ds(c * _CHUNK, _CHUNK)]
            for jj in range(_CHUNK):
                j = c * _CHUNK + jj
                pltpu.async_copy(
                    table_hbm.at[pl.ds(vec[jj], 1)],
                    rows_v.at[pl.ds(j, 1)],
                    sem_r,
                )
            return carry

        lax.fori_loop(0, n_chunks, fire, 0)
        pltpu.make_async_copy(
            table_hbm.at[pl.ds(0, b_per_w)], rows_v, sem_r
        ).wait()
        pltpu.sync_copy(rows_v, out_hbm.at[pl.ds(base, b_per_w)])

    return gather


def kernel(embeddings, indices):
    (B,) = indices.shape
    _, D = embeddings.shape
    b_per_w = B // _NUM_WORKERS
    gather = _build_gather(B, D, embeddings.dtype, b_per_w)
    return gather(embeddings, indices.astype(jnp.int32))


# speedup vs baseline: 3.1360x; 1.0001x over previous
import functools

import jax
import jax.numpy as jnp
from jax import lax
from jax.experimental import pallas as pl
from jax.experimental.pallas import tpu as pltpu
from jax.experimental.pallas import tpu_sc as plsc

_NUM_CORES = 2
_NUM_SUBCORES = 16
_NUM_WORKERS = _NUM_CORES * _NUM_SUBCORES

_CHUNK = 16


def _build_gather(B, D, dtype, b_per_w):
    mesh = plsc.VectorSubcoreMesh(core_axis_name="c", subcore_axis_name="s")
    n_chunks = b_per_w // _CHUNK

    @functools.partial(
        pl.kernel,
        out_type=jax.ShapeDtypeStruct((B, D), dtype),
        mesh=mesh,
        compiler_params=pltpu.CompilerParams(use_tc_tiling_on_sc=True),
        scratch_types=[
            pltpu.VMEM((b_per_w,), jnp.int32),
            pltpu.VMEM((b_per_w, D), dtype),
            pltpu.SemaphoreType.DMA,
        ],
    )
    def gather(table_hbm, idx_hbm, out_hbm, idx_v, rows_v, sem_r):
        wid = lax.axis_index("s") * _NUM_CORES + lax.axis_index("c")
        base = wid * b_per_w
        pltpu.sync_copy(idx_hbm.at[pl.ds(base, b_per_w)], idx_v)

        def fire(c, carry):
            vec = idx_v[pl.ds(c * _CHUNK, _CHUNK)]
            for jj in range(_CHUNK):
                j = c * _CHUNK + jj
                pltpu.async_copy(
                    table_hbm.at[pl.ds(vec[jj], 1)],
                    rows_v.at[pl.ds(j, 1)],
                    sem_r,
                )
            return carry

        lax.fori_loop(0, n_chunks, fire, 0)
        pltpu.make_async_copy(
            table_hbm.at[pl.ds(0, b_per_w)], rows_v, sem_r
        ).wait()
        pltpu.sync_copy(rows_v, out_hbm.at[pl.ds(base, b_per_w)])

    return gather


def kernel(embeddings, indices):
    (B,) = indices.shape
    _, D = embeddings.shape
    b_per_w = B // _NUM_WORKERS
    gather = _build_gather(B, D, embeddings.dtype, b_per_w)
    return gather(embeddings, indices.astype(jnp.int32))


# speedup vs baseline: 3.6520x; 1.1645x over previous
import functools

import jax
import jax.numpy as jnp
from jax import lax
from jax.experimental import pallas as pl
from jax.experimental.pallas import tpu as pltpu
from jax.experimental.pallas import tpu_sc as plsc

_NUM_CORES = 2
_NUM_SUBCORES = 16
_NUM_WORKERS = _NUM_CORES * _NUM_SUBCORES

_LANES = 16
_UNROLL = 8


def _build_gather(D, V, B, dtype):
    mesh = plsc.VectorSubcoreMesh(core_axis_name="c", subcore_axis_name="s")
    n_dim_rounds = -(-D // _NUM_WORKERS)
    step = _LANES * _UNROLL
    n_iters = B // step

    blk_sz = 2048
    n_blocks = B // blk_sz
    iters_per_blk = blk_sz // step

    @functools.partial(
        pl.kernel,
        out_type=jax.ShapeDtypeStruct((D, B), dtype),
        mesh=mesh,
        compiler_params=pltpu.CompilerParams(needs_layout_passes=False),
        scratch_types=[
            pltpu.VMEM((V,), dtype),
            pltpu.VMEM((B,), jnp.int32),
            pltpu.VMEM((blk_sz,), dtype),
            pltpu.VMEM((blk_sz,), dtype),
            pltpu.SemaphoreType.DMA,
        ],
    )
    def gather(tableT_hbm, idx_hbm, outT_hbm, row_v, idx_v, out_a, out_b, sem_o):
        wid = lax.axis_index("s") * _NUM_CORES + lax.axis_index("c")
        pltpu.sync_copy(idx_hbm, idx_v)
        for k in range(n_dim_rounds):
            d = wid + _NUM_WORKERS * k

            @pl.when(d < D)
            def _():
                pltpu.sync_copy(tableT_hbm.at[d], row_v)
                for blk in range(n_blocks):
                    buf = out_a if blk % 2 == 0 else out_b
                    if blk >= 2:
                        pltpu.make_async_copy(
                            buf,
                            outT_hbm.at[d, pl.ds((blk - 2) * blk_sz, blk_sz)],
                            sem_o,
                        ).wait()

                    def sweep(i, carry, blk=blk, buf=buf):
                        base = blk * blk_sz + i * step
                        for u in range(_UNROLL):
                            o = base + u * _LANES
                            ids = idx_v[pl.ds(o, _LANES)]
                            buf[
                                pl.ds(i * step + u * _LANES, _LANES)
                            ] = plsc.load_gather(row_v, [ids])
                        return carry

                    lax.fori_loop(0, iters_per_blk, sweep, 0)
                    pltpu.async_copy(
                        buf,
                        outT_hbm.at[d, pl.ds(blk * blk_sz, blk_sz)],
                        sem_o,
                    )
                for blk in (n_blocks - 2, n_blocks - 1):
                    pltpu.make_async_copy(
                        out_a if blk % 2 == 0 else out_b,
                        outT_hbm.at[d, pl.ds(blk * blk_sz, blk_sz)],
                        sem_o,
                    ).wait()

    return gather


def kernel(embeddings, indices):
    (B,) = indices.shape
    V, D = embeddings.shape
    gather = _build_gather(D, V, B, embeddings.dtype)
    outT = gather(embeddings.T, indices.astype(jnp.int32))
    return outT.T


# speedup vs baseline: 4.6060x; 1.2612x over previous
import functools

import jax
import jax.numpy as jnp
from jax import lax
from jax.experimental import pallas as pl
from jax.experimental.pallas import tpu as pltpu
from jax.experimental.pallas import tpu_sc as plsc

_NUM_CORES = 2
_NUM_SUBCORES = 16
_NUM_WORKERS = _NUM_CORES * _NUM_SUBCORES

_LANES = 16
_UNROLL = 8


def _build_gather(D, V, B, dtype):
    mesh = plsc.VectorSubcoreMesh(core_axis_name="c", subcore_axis_name="s")
    n_dim_rounds = -(-D // _NUM_WORKERS)
    step = _LANES * _UNROLL
    n_iters = B // step

    blk_sz = 2048
    n_blocks = B // blk_sz
    iters_per_blk = blk_sz // step

    @functools.partial(
        pl.kernel,
        out_type=jax.ShapeDtypeStruct((D, B), dtype),
        mesh=mesh,
        compiler_params=pltpu.CompilerParams(needs_layout_passes=False),
        scratch_types=[
            pltpu.VMEM((V,), dtype),
            pltpu.VMEM((B,), jnp.int32),
            pltpu.VMEM((blk_sz,), dtype),
            pltpu.VMEM((blk_sz,), dtype),
            pltpu.SemaphoreType.DMA,
        ],
    )
    def gather(tableT_hbm, idx_hbm, outT_hbm, row_v, idx_v, out_a, out_b, sem_o):
        wid = lax.axis_index("s") * _NUM_CORES + lax.axis_index("c")
        pltpu.sync_copy(idx_hbm, idx_v)
        for k in range(n_dim_rounds):
            d = wid + _NUM_WORKERS * k

            @pl.when(d < D)
            def _():
                pltpu.sync_copy(tableT_hbm.at[d], row_v)
                for blk in range(n_blocks):
                    buf = out_a if blk % 2 == 0 else out_b
                    if blk >= 2:
                        pltpu.make_async_copy(
                            buf,
                            outT_hbm.at[d, pl.ds((blk - 2) * blk_sz, blk_sz)],
                            sem_o,
                        ).wait()

                    @plsc.parallel_loop(
                        0, blk_sz, step=_LANES, unroll=_UNROLL
                    )
                    def sweep(o, blk=blk, buf=buf):
                        ids = idx_v[pl.ds(blk * blk_sz + o, _LANES)]
                        buf[pl.ds(o, _LANES)] = plsc.load_gather(
                            row_v, [ids]
                        )
                    pltpu.async_copy(
                        buf,
                        outT_hbm.at[d, pl.ds(blk * blk_sz, blk_sz)],
                        sem_o,
                    )
                for blk in (n_blocks - 2, n_blocks - 1):
                    pltpu.make_async_copy(
                        out_a if blk % 2 == 0 else out_b,
                        outT_hbm.at[d, pl.ds(blk * blk_sz, blk_sz)],
                        sem_o,
                    ).wait()

    return gather


def kernel(embeddings, indices):
    (B,) = indices.shape
    V, D = embeddings.shape
    gather = _build_gather(D, V, B, embeddings.dtype)
    outT = gather(embeddings.T, indices.astype(jnp.int32))
    return outT.T


# speedup vs baseline: 4.6426x; 1.0079x over previous
import functools

import jax
import jax.numpy as jnp
from jax import lax
from jax.experimental import pallas as pl
from jax.experimental.pallas import tpu as pltpu
from jax.experimental.pallas import tpu_sc as plsc

_NUM_CORES = 2
_NUM_SUBCORES = 16
_NUM_WORKERS = _NUM_CORES * _NUM_SUBCORES

_LANES = 16
_UNROLL = 16


def _build_gather(D, V, B, dtype):
    mesh = plsc.VectorSubcoreMesh(core_axis_name="c", subcore_axis_name="s")
    n_dim_rounds = -(-D // _NUM_WORKERS)
    step = _LANES * _UNROLL
    n_iters = B // step

    blk_sz = 4096
    n_blocks = B // blk_sz
    iters_per_blk = blk_sz // step

    @functools.partial(
        pl.kernel,
        out_type=jax.ShapeDtypeStruct((D, B), dtype),
        mesh=mesh,
        compiler_params=pltpu.CompilerParams(needs_layout_passes=False),
        scratch_types=[
            pltpu.VMEM((V,), dtype),
            pltpu.VMEM((B,), jnp.int32),
            pltpu.VMEM((blk_sz,), dtype),
            pltpu.VMEM((blk_sz,), dtype),
            pltpu.SemaphoreType.DMA,
        ],
    )
    def gather(tableT_hbm, idx_hbm, outT_hbm, row_v, idx_v, out_a, out_b, sem_o):
        wid = lax.axis_index("s") * _NUM_CORES + lax.axis_index("c")
        pltpu.sync_copy(idx_hbm, idx_v)
        for k in range(n_dim_rounds):
            d = wid + _NUM_WORKERS * k

            @pl.when(d < D)
            def _():
                pltpu.sync_copy(tableT_hbm.at[d], row_v)
                for blk in range(n_blocks):
                    buf = out_a if blk % 2 == 0 else out_b
                    if blk >= 2:
                        pltpu.make_async_copy(
                            buf,
                            outT_hbm.at[d, pl.ds((blk - 2) * blk_sz, blk_sz)],
                            sem_o,
                        ).wait()

                    @plsc.parallel_loop(
                        0, blk_sz, step=_LANES, unroll=_UNROLL
                    )
                    def sweep(o, blk=blk, buf=buf):
                        ids = idx_v[pl.ds(blk * blk_sz + o, _LANES)]
                        buf[pl.ds(o, _LANES)] = plsc.load_gather(
                            row_v, [ids]
                        )
                    pltpu.async_copy(
                        buf,
                        outT_hbm.at[d, pl.ds(blk * blk_sz, blk_sz)],
                        sem_o,
                    )
                for blk in (n_blocks - 2, n_blocks - 1):
                    pltpu.make_async_copy(
                        out_a if blk % 2 == 0 else out_b,
                        outT_hbm.at[d, pl.ds(blk * blk_sz, blk_sz)],
                        sem_o,
                    ).wait()

    return gather


def kernel(embeddings, indices):
    (B,) = indices.shape
    V, D = embeddings.shape
    gather = _build_gather(D, V, B, embeddings.dtype)
    outT = gather(embeddings.T, indices.astype(jnp.int32))
    return outT.T
